# baseline (speedup 1.0000x reference)
import functools

import jax
import jax.numpy as jnp
from jax import lax
from jax.experimental import pallas as pl
from jax.experimental.pallas import tpu as pltpu
from jax.experimental.pallas import tpu_sc as plsc

_D_ID = 32
_D_T = 32
_D_OUT = _D_ID + _D_T


def _table_body(id_ref, t_ref, w1_ref, b1_ref, w2t_ref, b2_ref, out_ref):
    t = t_ref[...]
    t_scale = jnp.max(t) + 1e-6
    ta = t / t_scale
    h = jnp.maximum(ta * w1_ref[...] + b1_ref[...], 0.0)
    e_t = jnp.dot(h, w2t_ref[...], preferred_element_type=jnp.float32)
    e_t = e_t + b2_ref[...]
    out_ref[:, :_D_ID] = id_ref[...]
    out_ref[:, _D_ID:] = e_t


def _build_table(id_emb, t_from_A, W1, b1, W2, b2):
    n = id_emb.shape[0]
    return pl.pallas_call(
        _table_body,
        out_shape=jax.ShapeDtypeStruct((n, _D_OUT), jnp.float32),
    )(
        id_emb,
        t_from_A.reshape(n, 1),
        W1.reshape(1, _D_T),
        b1.reshape(1, _D_T),
        W2.T,
        b2.reshape(1, _D_T),
    )


_NC = 2
_NS = 16
_NW = _NC * _NS
_LANES = 16


def _gather_body(n_stations, L, table_hbm, idsT_hbm, out_hbm,
                 tab_v, ids_v, blk_v, wsems):
    wid = lax.axis_index("s") * _NC + lax.axis_index("c")
    bt = wid

    pltpu.sync_copy(table_hbm, tab_v.at[:, : _D_OUT])
    pltpu.sync_copy(idsT_hbm.at[:, pl.ds(bt * 128, 128)], ids_v)

    def fill(l, b):
        for g in range(8):
            ids16 = ids_v[l, pl.ds(g * _LANES, _LANES)]
            col = jnp.zeros((_LANES,), jnp.int32)
            one = jnp.ones((_LANES,), jnp.int32)
            for f in range(_D_OUT):
                v = plsc.load_gather(tab_v, [ids16, col])
                blk_v[b, 0, f // 8, 0, f % 8, pl.ds(g * _LANES, _LANES)] = v
                if f + 1 < _D_OUT:
                    col = col + one

    def wb_start(l, b):
        pltpu.async_copy(
            blk_v.at[b], out_hbm.at[pl.ds(l, 1), :, pl.ds(bt, 1)],
            wsems.at[b])

    def wb_wait(l, b):
        pltpu.make_async_copy(
            blk_v.at[b], out_hbm.at[pl.ds(l, 1), :, pl.ds(bt, 1)],
            wsems.at[b]).wait()

    @pl.loop(0, L, step=2)
    def _pair(l):
        for b in range(2):
            ll = l + b

            @pl.when(ll >= 2)
            def _():
                wb_wait(ll - 2, b)

            fill(ll, b)
            wb_start(ll, b)

    wb_wait(L - 2, 0)
    wb_wait(L - 1, 1)


def _gather(table, idsT):
    n, d = table.shape
    L = idsT.shape[0]
    assert d == _D_OUT and idsT.shape[1] == _NW * 128 and L % 2 == 0
    mesh = plsc.VectorSubcoreMesh(core_axis_name="c", subcore_axis_name="s")
    k = pl.kernel(
        functools.partial(_gather_body, n, L),
        out_type=jax.ShapeDtypeStruct((L, 8, _NW, 8, 128), jnp.float32),
        mesh=mesh,
        scratch_types=[
            pltpu.VMEM((n, _D_OUT + 1), jnp.float32),
            pltpu.VMEM((L, 128), jnp.int32),
            pltpu.VMEM((2, 1, 8, 1, 8, 128), jnp.float32),
            pltpu.SemaphoreType.DMA((2,)),
        ],
        compiler_params=pltpu.CompilerParams(
            use_tc_tiling_on_sc=False, needs_layout_passes=False),
    )
    return k(table, idsT)


@jax.jit
def kernel(station_ids, id_emb, t_from_A, W1, b1, W2, b2):
    B, L = station_ids.shape
    table = _build_table(id_emb, t_from_A, W1, b1, W2, b2)
    out5 = _gather(table, station_ids.T)
    return out5.transpose(2, 4, 0, 1, 3).reshape(B, L, _D_OUT)


# speedup vs baseline: 1.8204x; 1.8204x over previous
import functools

import jax
import jax.numpy as jnp
from jax import lax
from jax.experimental import pallas as pl
from jax.experimental.pallas import tpu as pltpu
from jax.experimental.pallas import tpu_sc as plsc

_D_ID = 32
_D_T = 32
_D_OUT = _D_ID + _D_T
_BPAD = 129


def _table_body(id_ref, t_ref, w1_ref, b1_ref, w2t_ref, b2_ref, out_ref):
    t = t_ref[...]
    t_scale = jnp.max(t) + 1e-6
    ta = t / t_scale
    h = jnp.maximum(ta * w1_ref[...] + b1_ref[...], 0.0)
    e_t = jnp.dot(h, w2t_ref[...], preferred_element_type=jnp.float32)
    e_t = e_t + b2_ref[...]
    out_ref[:, :_D_ID] = id_ref[...]
    out_ref[:, _D_ID:] = e_t


def _build_table(id_emb, t_from_A, W1, b1, W2, b2):
    n = id_emb.shape[0]
    return pl.pallas_call(
        _table_body,
        out_shape=jax.ShapeDtypeStruct((n, _D_OUT), jnp.float32),
    )(
        id_emb,
        t_from_A.reshape(n, 1),
        W1.reshape(1, _D_T),
        b1.reshape(1, _D_T),
        W2.T,
        b2.reshape(1, _D_T),
    )


_NC = 2
_NS = 16
_NW = _NC * _NS
_LANES = 16


def _gather_body(n_stations, L, table_hbm, idsT_hbm, out_hbm,
                 table_sh, ids_v, rows_v, blk_v, gsems, wsems):
    wid = lax.axis_index("s") * _NC + lax.axis_index("c")
    bt = wid

    @pl.when(lax.axis_index("s") == 0)
    def _():
        pltpu.sync_copy(table_hbm, table_sh)

    pltpu.sync_copy(idsT_hbm.at[:, pl.ds(bt * 128, 128)], ids_v)
    plsc.subcore_barrier()

    def rg_start(l, rb):
        pltpu.async_copy(
            table_sh.at[ids_v.at[l]], rows_v.at[rb], gsems.at[rb])

    def rg_wait(l, rb):
        pltpu.make_async_copy(
            table_sh.at[ids_v.at[l]], rows_v.at[rb], gsems.at[rb]).wait()

    iota = lax.broadcasted_iota(jnp.int32, (_LANES,), 0)
    one = jnp.ones((_LANES,), jnp.int32)
    ft_vecs = [(iota + q * _LANES) // 8 for q in range(4)]
    fi_vecs = [(iota + q * _LANES) % 8 for q in range(4)]

    def transpose(rb, b):
        b_vec = jnp.zeros((_LANES,), jnp.int32)
        for bb in range(128):
            for q in range(4):
                v = rows_v[rb, bb, pl.ds(q * _LANES, _LANES)]
                plsc.store_scatter(
                    blk_v.at[b], [ft_vecs[q], fi_vecs[q], b_vec], v)
            if bb + 1 < 128:
                b_vec = b_vec + one

    def wb_start(l, b):
        pltpu.async_copy(
            blk_v.at[b, :, :, pl.ds(0, 128)], out_hbm.at[l, :, bt],
            wsems.at[b])

    def wb_wait(l, b):
        pltpu.make_async_copy(
            blk_v.at[b, :, :, pl.ds(0, 128)], out_hbm.at[l, :, bt],
            wsems.at[b]).wait()

    rg_start(0, 0)

    @pl.loop(0, L, step=2)
    def _pair(l):
        for b in range(2):
            ll = l + b

            rg_wait(ll, b)

            def _next(ll=ll, b=b):
                rg_start(ll + 1, 1 - b)

            if b == 0:
                _next()
            else:
                pl.when(ll + 1 < L)(_next)

            @pl.when(ll >= 2)
            def _():
                wb_wait(ll - 2, b)

            transpose(b, b)
            wb_start(ll, b)

    wb_wait(L - 2, 0)
    wb_wait(L - 1, 1)


def _gather(table, idsT):
    n, d = table.shape
    L = idsT.shape[0]
    assert d == _D_OUT and idsT.shape[1] == _NW * 128 and L % 2 == 0
    mesh = plsc.VectorSubcoreMesh(core_axis_name="c", subcore_axis_name="s")
    k = pl.kernel(
        functools.partial(_gather_body, n, L),
        out_type=jax.ShapeDtypeStruct((L, 8, _NW, 8, 128), jnp.float32),
        mesh=mesh,
        scratch_types=[
            pltpu.VMEM_SHARED((n, _D_OUT), jnp.float32),
            pltpu.VMEM((L, 128), jnp.int32),
            pltpu.VMEM((2, 128, _D_OUT), jnp.float32),
            pltpu.VMEM((2, 8, 8, _BPAD), jnp.float32),
            pltpu.SemaphoreType.DMA((2,)),
            pltpu.SemaphoreType.DMA((2,)),
        ],
        compiler_params=pltpu.CompilerParams(
            use_tc_tiling_on_sc=False, needs_layout_passes=False),
    )
    return k(table, idsT)


@jax.jit
def kernel(station_ids, id_emb, t_from_A, W1, b1, W2, b2):
    B, L = station_ids.shape
    table = _build_table(id_emb, t_from_A, W1, b1, W2, b2)
    out5 = _gather(table, station_ids.T)
    return out5.transpose(2, 4, 0, 1, 3).reshape(B, L, _D_OUT)


# speedup vs baseline: 5.8002x; 3.1863x over previous
import functools

import jax
import jax.numpy as jnp
from jax import lax
from jax.experimental import pallas as pl
from jax.experimental.pallas import tpu as pltpu
from jax.experimental.pallas import tpu_sc as plsc

_D_ID = 32
_D_T = 32
_D_OUT = _D_ID + _D_T
_BPAD = 129


def _table_body(id_ref, t_ref, w1_ref, b1_ref, w2t_ref, b2_ref, out_ref):
    t = t_ref[...]
    t_scale = jnp.max(t) + 1e-6
    ta = t / t_scale
    h = jnp.maximum(ta * w1_ref[...] + b1_ref[...], 0.0)
    e_t = jnp.dot(h, w2t_ref[...], preferred_element_type=jnp.float32)
    e_t = e_t + b2_ref[...]
    out_ref[:, :_D_ID] = id_ref[...]
    out_ref[:, _D_ID:] = e_t


def _build_table(id_emb, t_from_A, W1, b1, W2, b2):
    n = id_emb.shape[0]
    return pl.pallas_call(
        _table_body,
        out_shape=jax.ShapeDtypeStruct((n, _D_OUT), jnp.float32),
    )(
        id_emb,
        t_from_A.reshape(n, 1),
        W1.reshape(1, _D_T),
        b1.reshape(1, _D_T),
        W2.T,
        b2.reshape(1, _D_T),
    )


_NC = 2
_NS = 16
_NW = _NC * _NS
_LANES = 16


def _gather_body(n_stations, L, table_hbm, idsT_hbm, out_hbm,
                 table_sh, ids_v, rows_v, blk_v, gsems, wsems):
    wid = lax.axis_index("s") * _NC + lax.axis_index("c")
    bt = wid

    @pl.when(lax.axis_index("s") == 0)
    def _():
        pltpu.sync_copy(table_hbm, table_sh)

    pltpu.sync_copy(idsT_hbm.at[:, pl.ds(bt * 128, 128)], ids_v)
    plsc.subcore_barrier()

    def rg_start(l, rb):
        pltpu.async_copy(
            table_sh.at[ids_v.at[l]], rows_v.at[rb], gsems.at[rb])

    def rg_wait(l, rb):
        pltpu.make_async_copy(
            table_sh.at[ids_v.at[l]], rows_v.at[rb], gsems.at[rb]).wait()

    iota = lax.broadcasted_iota(jnp.int32, (_LANES,), 0)
    one = jnp.ones((_LANES,), jnp.int32)
    ft_vecs = [(iota + q * _LANES) // 8 for q in range(4)]
    fi_vecs = [(iota + q * _LANES) % 8 for q in range(4)]

    def transpose(rb, b):
        @plsc.parallel_loop(0, 128, 1, unroll=8)
        def _t(bb):
            b_vec = jnp.zeros((_LANES,), jnp.int32) + bb
            for q in range(4):
                v = rows_v[rb, bb, pl.ds(q * _LANES, _LANES)]
                plsc.store_scatter(
                    blk_v.at[b], [ft_vecs[q], fi_vecs[q], b_vec], v)

    def wb_start(l, b):
        pltpu.async_copy(
            blk_v.at[b, :, :, pl.ds(0, 128)], out_hbm.at[l, :, bt],
            wsems.at[b])

    def wb_wait(l, b):
        pltpu.make_async_copy(
            blk_v.at[b, :, :, pl.ds(0, 128)], out_hbm.at[l, :, bt],
            wsems.at[b]).wait()

    rg_start(0, 0)

    @pl.loop(0, L, step=2)
    def _pair(l):
        for b in range(2):
            ll = l + b

            rg_wait(ll, b)

            def _next(ll=ll, b=b):
                rg_start(ll + 1, 1 - b)

            if b == 0:
                _next()
            else:
                pl.when(ll + 1 < L)(_next)

            @pl.when(ll >= 2)
            def _():
                wb_wait(ll - 2, b)

            transpose(b, b)
            wb_start(ll, b)

    wb_wait(L - 2, 0)
    wb_wait(L - 1, 1)


def _gather(table, idsT):
    n, d = table.shape
    L = idsT.shape[0]
    assert d == _D_OUT and idsT.shape[1] == _NW * 128 and L % 2 == 0
    mesh = plsc.VectorSubcoreMesh(core_axis_name="c", subcore_axis_name="s")
    k = pl.kernel(
        functools.partial(_gather_body, n, L),
        out_type=jax.ShapeDtypeStruct((L, 8, _NW, 8, 128), jnp.float32),
        mesh=mesh,
        scratch_types=[
            pltpu.VMEM_SHARED((n, _D_OUT), jnp.float32),
            pltpu.VMEM((L, 128), jnp.int32),
            pltpu.VMEM((2, 128, _D_OUT), jnp.float32),
            pltpu.VMEM((2, 8, 8, _BPAD), jnp.float32),
            pltpu.SemaphoreType.DMA((2,)),
            pltpu.SemaphoreType.DMA((2,)),
        ],
        compiler_params=pltpu.CompilerParams(
            use_tc_tiling_on_sc=False, needs_layout_passes=False),
    )
    return k(table, idsT)


@jax.jit
def kernel(station_ids, id_emb, t_from_A, W1, b1, W2, b2):
    B, L = station_ids.shape
    table = _build_table(id_emb, t_from_A, W1, b1, W2, b2)
    out5 = _gather(table, station_ids.T)
    return out5.transpose(2, 4, 0, 1, 3).reshape(B, L, _D_OUT)


# speedup vs baseline: 6.2397x; 1.0758x over previous
import functools

import jax
import jax.numpy as jnp
from jax import lax
from jax.experimental import pallas as pl
from jax.experimental.pallas import tpu as pltpu
from jax.experimental.pallas import tpu_sc as plsc

_D_ID = 32
_D_T = 32
_D_OUT = _D_ID + _D_T
_BPAD = 129


def _table_body(id_ref, t_ref, w1_ref, b1_ref, w2t_ref, b2_ref, out_ref):
    t = t_ref[...]
    t_scale = jnp.max(t) + 1e-6
    ta = t / t_scale
    h = jnp.maximum(ta * w1_ref[...] + b1_ref[...], 0.0)
    e_t = jnp.dot(h, w2t_ref[...], preferred_element_type=jnp.float32)
    e_t = e_t + b2_ref[...]
    out_ref[:, :_D_ID] = id_ref[...]
    out_ref[:, _D_ID:] = e_t


def _build_table(id_emb, t_from_A, W1, b1, W2, b2):
    n = id_emb.shape[0]
    return pl.pallas_call(
        _table_body,
        out_shape=jax.ShapeDtypeStruct((n, _D_OUT), jnp.float32),
    )(
        id_emb,
        t_from_A.reshape(n, 1),
        W1.reshape(1, _D_T),
        b1.reshape(1, _D_T),
        W2.T,
        b2.reshape(1, _D_T),
    )


_NC = 2
_NS = 16
_NW = _NC * _NS
_LANES = 16
_NBUF = 4


def _gather_body(n_stations, L, table_hbm, idsT_hbm, out_hbm,
                 table_sh, ids_v, rows_v, blk_v, gsems, wsems):
    wid = lax.axis_index("s") * _NC + lax.axis_index("c")
    bt = wid

    @pl.when(lax.axis_index("s") == 0)
    def _():
        pltpu.sync_copy(table_hbm, table_sh)

    pltpu.sync_copy(idsT_hbm.at[:, pl.ds(bt * 128, 128)], ids_v)
    plsc.subcore_barrier()

    def rg_start(l, rb):
        pltpu.async_copy(
            table_sh.at[ids_v.at[l]], rows_v.at[rb], gsems.at[rb])

    def rg_wait(l, rb):
        pltpu.make_async_copy(
            table_sh.at[ids_v.at[l]], rows_v.at[rb], gsems.at[rb]).wait()

    iota = lax.broadcasted_iota(jnp.int32, (_LANES,), 0)
    one = jnp.ones((_LANES,), jnp.int32)
    ft_vecs = [(iota + q * _LANES) // 8 for q in range(4)]
    fi_vecs = [(iota + q * _LANES) % 8 for q in range(4)]

    def transpose(rb, b):
        @plsc.parallel_loop(0, 128, 1, unroll=8)
        def _t(bb):
            b_vec = jnp.zeros((_LANES,), jnp.int32) + bb
            for q in range(4):
                v = rows_v[rb, bb, pl.ds(q * _LANES, _LANES)]
                plsc.store_scatter(
                    blk_v.at[b], [ft_vecs[q], fi_vecs[q], b_vec], v)

    def wb_start(l, b):
        pltpu.async_copy(
            blk_v.at[b, :, :, pl.ds(0, 128)], out_hbm.at[l, :, bt],
            wsems.at[b])

    def wb_wait(l, b):
        pltpu.make_async_copy(
            blk_v.at[b, :, :, pl.ds(0, 128)], out_hbm.at[l, :, bt],
            wsems.at[b]).wait()

    for i in range(_NBUF - 1):
        rg_start(i, i)

    @pl.loop(0, L, step=_NBUF)
    def _grp(l):
        for b in range(_NBUF):
            ll = l + b

            rg_wait(ll, b)

            def _next(ll=ll, b=b):
                rg_start(ll + _NBUF - 1, (b + _NBUF - 1) % _NBUF)

            if b == 0:
                _next()
            else:
                pl.when(ll + _NBUF - 1 < L)(_next)

            @pl.when(ll >= _NBUF)
            def _():
                wb_wait(ll - _NBUF, b)

            transpose(b, b)
            wb_start(ll, b)

    for b in range(_NBUF):
        wb_wait(L - _NBUF + b, b)


def _gather(table, idsT):
    n, d = table.shape
    L = idsT.shape[0]
    assert d == _D_OUT and idsT.shape[1] == _NW * 128 and L % _NBUF == 0
    mesh = plsc.VectorSubcoreMesh(core_axis_name="c", subcore_axis_name="s")
    k = pl.kernel(
        functools.partial(_gather_body, n, L),
        out_type=jax.ShapeDtypeStruct((L, 8, _NW, 8, 128), jnp.float32),
        mesh=mesh,
        scratch_types=[
            pltpu.VMEM_SHARED((n, _D_OUT), jnp.float32),
            pltpu.VMEM((L, 128), jnp.int32),
            pltpu.VMEM((_NBUF, 128, _D_OUT), jnp.float32),
            pltpu.VMEM((_NBUF, 8, 8, _BPAD), jnp.float32),
            pltpu.SemaphoreType.DMA((_NBUF,)),
            pltpu.SemaphoreType.DMA((_NBUF,)),
        ],
        compiler_params=pltpu.CompilerParams(
            use_tc_tiling_on_sc=False, needs_layout_passes=False),
    )
    return k(table, idsT)


@jax.jit
def kernel(station_ids, id_emb, t_from_A, W1, b1, W2, b2):
    B, L = station_ids.shape
    table = _build_table(id_emb, t_from_A, W1, b1, W2, b2)
    out5 = _gather(table, station_ids.T)
    return out5.transpose(2, 4, 0, 1, 3).reshape(B, L, _D_OUT)


# speedup vs baseline: 6.6054x; 1.0586x over previous
import functools

import jax
import jax.numpy as jnp
from jax import lax
from jax.experimental import pallas as pl
from jax.experimental.pallas import tpu as pltpu
from jax.experimental.pallas import tpu_sc as plsc

_D_ID = 32
_D_T = 32
_D_OUT = _D_ID + _D_T
_BPAD = 129


def _table_body(idT_ref, t_ref, w1_ref, b1_ref, w2_ref, b2_ref, out_ref):
    t = t_ref[...].reshape(-1, 1)
    t_scale = jnp.max(t) + 1e-6
    ta = t / t_scale
    h = jnp.maximum(ta * w1_ref[...] + b1_ref[...], 0.0)
    e_t = jnp.dot(h, w2_ref[...].T, preferred_element_type=jnp.float32)
    e_t = e_t + b2_ref[...]
    out_ref[:, :_D_ID] = idT_ref[...].T
    out_ref[:, _D_ID:] = e_t


def _build_table(id_emb, t_from_A, W1, b1, W2, b2):
    n = id_emb.shape[0]
    return pl.pallas_call(
        _table_body,
        out_shape=jax.ShapeDtypeStruct((n, _D_OUT), jnp.float32),
    )(
        id_emb.T,
        t_from_A,
        W1.reshape(1, _D_T),
        b1.reshape(1, _D_T),
        W2,
        b2.reshape(1, _D_T),
    )


_NC = 2
_NS = 16
_NW = _NC * _NS
_LANES = 16
_NBUF = 4


def _gather_body(n_stations, L, table_hbm, ids4_hbm, out_hbm,
                 table_sh, ids_v, rows_v, blk_v, gsems, wsems):
    wid = lax.axis_index("s") * _NC + lax.axis_index("c")
    bt = wid

    @pl.when(lax.axis_index("s") == 0)
    def _():
        pltpu.sync_copy(table_hbm, table_sh)

    pltpu.sync_copy(ids4_hbm.at[:, bt], ids_v)
    plsc.subcore_barrier()

    def rg_start(l, rb):
        pltpu.async_copy(
            table_sh.at[ids_v.at[l // 8, l % 8]], rows_v.at[rb],
            gsems.at[rb])

    def rg_wait(l, rb):
        pltpu.make_async_copy(
            table_sh.at[ids_v.at[l // 8, l % 8]], rows_v.at[rb],
            gsems.at[rb]).wait()

    iota = lax.broadcasted_iota(jnp.int32, (_LANES,), 0)
    one = jnp.ones((_LANES,), jnp.int32)
    ft_vecs = [(iota + q * _LANES) // 8 for q in range(4)]
    fi_vecs = [(iota + q * _LANES) % 8 for q in range(4)]

    def transpose(rb, b):
        @plsc.parallel_loop(0, 128, 1, unroll=8)
        def _t(bb):
            b_vec = jnp.zeros((_LANES,), jnp.int32) + bb
            for q in range(4):
                v = rows_v[rb, bb, pl.ds(q * _LANES, _LANES)]
                plsc.store_scatter(
                    blk_v.at[b], [ft_vecs[q], fi_vecs[q], b_vec], v)

    def wb_start(l, b):
        pltpu.async_copy(
            blk_v.at[b, :, :, pl.ds(0, 128)], out_hbm.at[l, :, bt],
            wsems.at[b])

    def wb_wait(l, b):
        pltpu.make_async_copy(
            blk_v.at[b, :, :, pl.ds(0, 128)], out_hbm.at[l, :, bt],
            wsems.at[b]).wait()

    for i in range(_NBUF - 1):
        rg_start(i, i)

    @pl.loop(0, L, step=_NBUF)
    def _grp(l):
        for b in range(_NBUF):
            ll = l + b

            rg_wait(ll, b)

            def _next(ll=ll, b=b):
                rg_start(ll + _NBUF - 1, (b + _NBUF - 1) % _NBUF)

            if b == 0:
                _next()
            else:
                pl.when(ll + _NBUF - 1 < L)(_next)

            @pl.when(ll >= _NBUF)
            def _():
                wb_wait(ll - _NBUF, b)

            transpose(b, b)
            wb_start(ll, b)

    for b in range(_NBUF):
        wb_wait(L - _NBUF + b, b)


def _gather(table, ids4):
    n, d = table.shape
    L = ids4.shape[0] * 8
    assert d == _D_OUT and ids4.shape[1] == _NW and L % _NBUF == 0
    mesh = plsc.VectorSubcoreMesh(core_axis_name="c", subcore_axis_name="s")
    k = pl.kernel(
        functools.partial(_gather_body, n, L),
        out_type=jax.ShapeDtypeStruct((L, 8, _NW, 8, 128), jnp.float32),
        mesh=mesh,
        scratch_types=[
            pltpu.VMEM_SHARED((n, _D_OUT), jnp.float32),
            pltpu.VMEM((L // 8, 8, 128), jnp.int32),
            pltpu.VMEM((_NBUF, 128, _D_OUT), jnp.float32),
            pltpu.VMEM((_NBUF, 8, 8, _BPAD), jnp.float32),
            pltpu.SemaphoreType.DMA((_NBUF,)),
            pltpu.SemaphoreType.DMA((_NBUF,)),
        ],
        compiler_params=pltpu.CompilerParams(
            use_tc_tiling_on_sc=False, needs_layout_passes=False),
    )
    return k(table, ids4)


@jax.jit
def kernel(station_ids, id_emb, t_from_A, W1, b1, W2, b2):
    B, L = station_ids.shape
    table = _build_table(id_emb, t_from_A, W1, b1, W2, b2)
    ids4 = station_ids.reshape(_NW, 128, L // 8, 8).transpose(2, 0, 3, 1)
    out5 = _gather(table, ids4)
    return out5.transpose(2, 4, 0, 1, 3).reshape(B, L, _D_OUT)
